# baseline (speedup 1.0000x reference)
import dataclasses
import functools

import jax
import jax.numpy as jnp
from jax import lax
from jax.experimental import pallas as pl
from jax.experimental.pallas import tpu as pltpu
from jax.experimental.pallas import tpu_sc as plsc

DIM = 32
N_CHUNKS = 4
NUM_CORES = 2
NUM_SUBCORES = 16
NUM_WORKERS = NUM_CORES * NUM_SUBCORES
LANES = 16

VOCAB = 1000000
PACK = 16
VROWS = VOCAB * N_CHUNKS // PACK

BATCH = 16384
BPW = BATCH // NUM_WORKERS
TOK_CHUNK = 128
N_TOK_CHUNKS = BPW // TOK_CHUNK


def _sc_body(x_hbm, table_hbm, w_hbm, h_hbm, out_hbm,
             x_v, xr_v, xo_v, h64_v, w64_v, hcol_v, wcol_v,
             vecs_v, out_v, sem):
    wid = lax.axis_index("s") * NUM_CORES + lax.axis_index("c")
    base = wid * BPW
    for j in range(N_TOK_CHUNKS):
        pltpu.sync_copy(x_hbm.at[pl.ds(base + j * TOK_CHUNK, TOK_CHUNK)],
                        x_v.at[j])
    for j in range(N_TOK_CHUNKS):
        @pl.loop(0, TOK_CHUNK // LANES)
        def _(t, j=j):
            s = pl.ds(t * LANES, LANES)
            xv = x_v[j, s]
            xr_v[j, s] = lax.shift_right_logical(xv, 2)
            xo_v[j, s] = lax.shift_left(jnp.bitwise_and(xv, 3), 2)

    lane = lax.iota(jnp.int32, LANES)
    for j in range(N_TOK_CHUNKS):
        row0 = base + j * TOK_CHUNK
        pltpu.async_copy(h_hbm.at[xr_v.at[j]], h64_v, sem).wait()
        pltpu.async_copy(w_hbm.at[xr_v.at[j]], w64_v, sem).wait()

        @pl.loop(0, TOK_CHUNK // LANES)
        def _(t, j=j):
            b0 = t * LANES
            rows = b0 + lane
            off = xo_v[j, pl.ds(b0, LANES)]
            for c in range(N_CHUNKS):
                hcol_v[c, pl.ds(b0, LANES)] = plsc.load_gather(
                    h64_v, [rows, off + c])
                wcol_v[c, pl.ds(b0, LANES)] = plsc.load_gather(
                    w64_v, [rows, off + c])

        for c in range(N_CHUNKS):
            pltpu.async_copy(
                table_hbm.at[hcol_v.at[c]],
                vecs_v.at[pl.ds(c * TOK_CHUNK, TOK_CHUNK)], sem).wait()

        @pl.loop(0, TOK_CHUNK)
        def _(b):
            brow = jnp.full((LANES,), b, jnp.int32)
            wv = plsc.load_gather(
                wcol_v, [jnp.zeros((LANES,), jnp.int32), brow])
            acc_lo = wv * vecs_v[b, pl.ds(0, LANES)]
            acc_hi = wv * vecs_v[b, pl.ds(LANES, LANES)]
            for c in range(1, N_CHUNKS):
                wv = plsc.load_gather(
                    wcol_v, [jnp.full((LANES,), c, jnp.int32), brow])
                r = c * TOK_CHUNK + b
                acc_lo = acc_lo + wv * vecs_v[r, pl.ds(0, LANES)]
                acc_hi = acc_hi + wv * vecs_v[r, pl.ds(LANES, LANES)]
            out_v[b, pl.ds(0, LANES)] = acc_lo
            out_v[b, pl.ds(LANES, LANES)] = acc_hi

        pltpu.sync_copy(out_v, out_hbm.at[pl.ds(row0, TOK_CHUNK)])


@functools.lru_cache(maxsize=1)
def _build_kernel():
    mesh = plsc.VectorSubcoreMesh(core_axis_name="c", subcore_axis_name="s")
    cp = pltpu.CompilerParams()
    fields = pltpu.CompilerParams.__dataclass_fields__
    if "needs_layout_passes" in fields:
        cp = dataclasses.replace(cp, needs_layout_passes=False)
    if "use_tc_tiling_on_sc" in fields:
        cp = dataclasses.replace(cp, use_tc_tiling_on_sc=False)
    return pl.kernel(
        _sc_body,
        out_type=jax.ShapeDtypeStruct((BATCH, DIM), jnp.float32),
        mesh=mesh,
        compiler_params=cp,
        scratch_types=[
            pltpu.VMEM((N_TOK_CHUNKS, TOK_CHUNK), jnp.int32),
            pltpu.VMEM((N_TOK_CHUNKS, TOK_CHUNK), jnp.int32),
            pltpu.VMEM((N_TOK_CHUNKS, TOK_CHUNK), jnp.int32),
            pltpu.VMEM((TOK_CHUNK, PACK), jnp.int32),
            pltpu.VMEM((TOK_CHUNK, PACK), jnp.float32),
            pltpu.VMEM((N_CHUNKS, TOK_CHUNK), jnp.int32),
            pltpu.VMEM((N_CHUNKS, TOK_CHUNK), jnp.float32),
            pltpu.VMEM((TOK_CHUNK * N_CHUNKS, DIM), jnp.float32),
            pltpu.VMEM((TOK_CHUNK, DIM), jnp.float32),
            pltpu.SemaphoreType.DMA,
        ],
    )


def kernel(x, table, weights, h):
    x = x.astype(jnp.int32)
    h16 = h.astype(jnp.int32).reshape(VROWS, PACK)
    w16 = weights.reshape(VROWS, PACK)
    return _build_kernel()(x, table, w16, h16)
